# baseline (speedup 1.0000x reference)
import functools

import jax
import jax.numpy as jnp
from jax import lax
from jax.experimental import pallas as pl
from jax.experimental.pallas import tpu as pltpu
from jax.experimental.pallas import tpu_sc as plsc

_LANES = 16


def _sc_embedding_bag(idx_d, w_d, idx_t, w_t, emb):
    NW, DPW = idx_d.shape
    _, NSUB, SUB = idx_t.shape
    V, H = emb.shape
    NG = H // _LANES
    mesh = plsc.VectorSubcoreMesh(core_axis_name="c", subcore_axis_name="s")
    NC = 2

    @functools.partial(
        pl.kernel,
        mesh=mesh,
        out_type=[
            jax.ShapeDtypeStruct((NW * DPW, H), jnp.float32),
            jax.ShapeDtypeStruct((NW, H), jnp.float32),
        ],
        scratch_types=[
            pltpu.VMEM((DPW,), jnp.int32),
            pltpu.VMEM((DPW,), jnp.float32),
            pltpu.VMEM((NSUB, SUB), jnp.int32),
            pltpu.VMEM((NSUB, SUB), jnp.float32),
            pltpu.VMEM((DPW, H), jnp.float32),
            pltpu.VMEM((SUB, H), jnp.float32),
            pltpu.VMEM((SUB, H), jnp.float32),
            pltpu.VMEM((1, H), jnp.float32),
            pltpu.SemaphoreType.DMA,
            pltpu.SemaphoreType.DMA,
        ],
    )
    def k(idx_d_hbm, w_d_hbm, idx_t_hbm, w_t_hbm, emb_hbm, bag_hbm, part_hbm,
          idxd_v, wd_v, idxt_v, wt_v, rowsd_v, rows0_v, rows1_v, acc_v,
          sem0, sem1):
        c = lax.axis_index("c")
        s = lax.axis_index("s")
        w = s * NC + c

        pltpu.sync_copy(idx_d_hbm.at[w], idxd_v)
        pltpu.sync_copy(w_d_hbm.at[w], wd_v)
        pltpu.sync_copy(idx_t_hbm.at[w], idxt_v)
        pltpu.sync_copy(w_t_hbm.at[w], wt_v)

        bufs = (rows0_v, rows1_v)
        sems = (sem0, sem1)
        pending = pltpu.async_copy(emb_hbm.at[idxt_v.at[0]], bufs[0], sems[0])

        pltpu.async_copy(emb_hbm.at[idxd_v], rowsd_v, sem1).wait()

        def dbody(rg, carry):
            wv = wd_v[pl.ds(rg * _LANES, _LANES)]
            for k in range(_LANES):
                r = rg * _LANES + k
                sw = wv[k]
                for g in range(NG):
                    sl = pl.ds(g * _LANES, _LANES)
                    rowsd_v[r, sl] = rowsd_v[r, sl] * sw
            return carry

        lax.fori_loop(0, DPW // _LANES, dbody, 0)
        pltpu.sync_copy(rowsd_v, bag_hbm.at[pl.ds(w * DPW, DPW)])

        def accumulate(j, buf, accs):
            def rbody(rg, a):
                wv = wt_v[j, pl.ds(rg * _LANES, _LANES)]
                for kk in range(_LANES):
                    r = rg * _LANES + kk
                    sw = wv[kk]
                    a = tuple(
                        a[g] + buf[r, pl.ds(g * _LANES, _LANES)] * sw
                        for g in range(NG)
                    )
                return a

            return lax.fori_loop(0, SUB // _LANES, rbody, accs)

        zero = jnp.zeros((_LANES,), jnp.float32)
        accs = (zero,) * NG
        for j in range(NSUB):
            nxt = None
            if j + 1 < NSUB:
                nxt = pltpu.async_copy(
                    emb_hbm.at[idxt_v.at[j + 1]],
                    bufs[(j + 1) % 2], sems[(j + 1) % 2])
            pending.wait()
            accs = accumulate(j, bufs[j % 2], accs)
            pending = nxt
        for g in range(NG):
            acc_v[0, pl.ds(g * _LANES, _LANES)] = accs[g]
        pltpu.sync_copy(acc_v, part_hbm.at[pl.ds(w, 1)])

    return k(idx_d, w_d, idx_t, w_t, emb)


def _tc_dense(bag, partials, l1_w, l1_b, l2_w, l2_b, l3_w, l3_b):
    B, H = bag.shape
    V = l3_w.shape[0]
    VB = 512
    nblk = V // VB
    VFULL = nblk * VB

    def body(bag_ref, part_ref, l1w_ref, l1b_ref, l2w_ref, l2b_ref,
             l3w_ref, l3b_ref, out_hbm, z_out, z_ref, buf_ref, sem):
        i = pl.program_id(0)
        p = lax.rem(i, 2)

        @pl.when(i == 0)
        def _():
            rows = lax.broadcasted_iota(jnp.int32, (B, 1), 0)
            tail = jnp.sum(part_ref[...], axis=0, keepdims=True)
            bagf = bag_ref[...] + jnp.where(rows == B - 1, 1.0, 0.0) * tail
            x = jnp.tanh(bagf)
            h = jnp.tanh(
                lax.dot_general(x, l1w_ref[...], (((1,), (1,)), ((), ())),
                                preferred_element_type=jnp.float32)
                + l1b_ref[...])
            z = jnp.tanh(
                lax.dot_general(h, l2w_ref[...], (((1,), (1,)), ((), ())),
                                preferred_element_type=jnp.float32)
                + l2b_ref[...])
            z_ref[...] = z
            z_out[...] = z

        def tile_copy(buf_slot, blk):
            return pltpu.make_async_copy(
                buf_ref.at[buf_slot],
                out_hbm.at[:, pl.ds(blk * VB, VB)],
                sem.at[buf_slot])

        @pl.when(i >= 2)
        def _():
            tile_copy(p, i - 2).wait()

        w_tile = l3w_ref[pl.ds(pl.multiple_of(i * VB, VB), VB), :]
        buf_ref[p] = (
            lax.dot_general(z_ref[...], w_tile, (((1,), (1,)), ((), ())),
                            preferred_element_type=jnp.float32)
            + l3b_ref[0])
        tile_copy(p, i).start()

        @pl.when(i == nblk - 1)
        def _():
            tile_copy(1 - p, i - 1).wait()
            tile_copy(p, i).wait()

    main, z = pl.pallas_call(
        body,
        grid=(nblk,),
        in_specs=[
            pl.BlockSpec((B, H), lambda i: (0, 0)),
            pl.BlockSpec(partials.shape, lambda i: (0, 0)),
            pl.BlockSpec(l1_w.shape, lambda i: (0, 0)),
            pl.BlockSpec((1, l1_w.shape[0]), lambda i: (0, 0)),
            pl.BlockSpec(l2_w.shape, lambda i: (0, 0)),
            pl.BlockSpec((1, l2_w.shape[0]), lambda i: (0, 0)),
            pl.BlockSpec((V, H), lambda i: (0, 0)),
            pl.BlockSpec((1, 1, VB), lambda i: (i, 0, 0)),
        ],
        out_specs=[
            pl.BlockSpec(memory_space=pl.ANY),
            pl.BlockSpec((B, H), lambda i: (0, 0)),
        ],
        out_shape=[
            jax.ShapeDtypeStruct((B, V), jnp.float32),
            jax.ShapeDtypeStruct((B, H), jnp.float32),
        ],
        scratch_shapes=[
            pltpu.VMEM((B, H), jnp.float32),
            pltpu.VMEM((2, B, VB), jnp.float32),
            pltpu.SemaphoreType.DMA((2,)),
        ],
        compiler_params=pltpu.CompilerParams(
            dimension_semantics=("arbitrary",)),
    )(bag, partials, l1_w, l1_b.reshape(1, -1), l2_w, l2_b.reshape(1, -1),
      l3_w, l3_b[:VFULL].reshape(nblk, 1, VB))

    PW = 512
    pblk = VFULL // PW
    l3w_tail = jnp.pad(l3_w[VFULL:], ((0, PW - (V - VFULL)), (0, 0)))
    l3b_tail = jnp.pad(l3_b[VFULL:], (0, PW - (V - VFULL))).reshape(1, PW)

    def patch_body(m_ref, z_ref, w_ref, b_ref, out_ref):
        del m_ref
        out_ref[...] = (
            lax.dot_general(z_ref[...], w_ref[...], (((1,), (1,)), ((), ())),
                            preferred_element_type=jnp.float32)
            + b_ref[...])

    return pl.pallas_call(
        patch_body,
        grid=(1,),
        in_specs=[
            pl.BlockSpec(memory_space=pl.ANY),
            pl.BlockSpec((B, H), lambda i: (0, 0)),
            pl.BlockSpec((PW, H), lambda i: (0, 0)),
            pl.BlockSpec((1, PW), lambda i: (0, 0)),
        ],
        out_specs=pl.BlockSpec((B, PW), lambda i: (0, pblk)),
        out_shape=jax.ShapeDtypeStruct((B, V), jnp.float32),
        input_output_aliases={0: 0},
    )(main, z, l3w_tail, l3b_tail)


def kernel(array, offsets, weights, emb_w, l1_w, l1_b, l2_w, l2_b, l3_w, l3_b):
    B = offsets.shape[0]
    N = array.shape[0]
    NW = 32
    DPW = B // NW
    tail = N - B
    per_w = tail // NW
    SUB = 112
    NSUB = per_w // SUB

    arr = array.astype(jnp.int32)
    idx_d = arr[:B].reshape(NW, DPW)
    w_d = weights[:B].reshape(NW, DPW)
    idx_t = arr[B:].reshape(NW, NSUB, SUB)
    w_t = weights[B:].reshape(NW, NSUB, SUB)

    bag, partials = _sc_embedding_bag(idx_d, w_d, idx_t, w_t, emb_w)
    return _tc_dense(bag, partials, l1_w, l1_b, l2_w, l2_b, l3_w, l3_b)


# speedup vs baseline: 1.1006x; 1.1006x over previous
import functools

import jax
import jax.numpy as jnp
from jax import lax
from jax.experimental import pallas as pl
from jax.experimental.pallas import tpu as pltpu
from jax.experimental.pallas import tpu_sc as plsc

_LANES = 16


def _sc_embedding_bag(idx_d, w_d, idx_t, w_t, emb):
    NW, DPW = idx_d.shape
    _, NSUB, SUB = idx_t.shape
    V, H = emb.shape
    NG = H // _LANES
    mesh = plsc.VectorSubcoreMesh(core_axis_name="c", subcore_axis_name="s")
    NC = 2

    @functools.partial(
        pl.kernel,
        mesh=mesh,
        out_type=[
            jax.ShapeDtypeStruct((NW * DPW, H), jnp.float32),
            jax.ShapeDtypeStruct((NW, H), jnp.float32),
        ],
        scratch_types=[
            pltpu.VMEM((DPW,), jnp.int32),
            pltpu.VMEM((DPW,), jnp.float32),
            pltpu.VMEM((NSUB, SUB), jnp.int32),
            pltpu.VMEM((NSUB, SUB), jnp.float32),
            pltpu.VMEM((DPW, H), jnp.float32),
            pltpu.VMEM((SUB, H), jnp.float32),
            pltpu.VMEM((SUB, H), jnp.float32),
            pltpu.VMEM((1, H), jnp.float32),
            pltpu.SemaphoreType.DMA,
            pltpu.SemaphoreType.DMA,
        ],
    )
    def k(idx_d_hbm, w_d_hbm, idx_t_hbm, w_t_hbm, emb_hbm, bag_hbm, part_hbm,
          idxd_v, wd_v, idxt_v, wt_v, rowsd_v, rows0_v, rows1_v, acc_v,
          sem0, sem1):
        c = lax.axis_index("c")
        s = lax.axis_index("s")
        w = s * NC + c

        pltpu.sync_copy(idx_d_hbm.at[w], idxd_v)
        pltpu.sync_copy(w_d_hbm.at[w], wd_v)
        pltpu.sync_copy(idx_t_hbm.at[w], idxt_v)
        pltpu.sync_copy(w_t_hbm.at[w], wt_v)

        bufs = (rows0_v, rows1_v)
        sems = (sem0, sem1)
        pending = pltpu.async_copy(emb_hbm.at[idxt_v.at[0]], bufs[0], sems[0])

        pltpu.async_copy(emb_hbm.at[idxd_v], rowsd_v, sem1).wait()

        def dbody(rg, carry):
            wv = wd_v[pl.ds(rg * _LANES, _LANES)]
            for k in range(_LANES):
                r = rg * _LANES + k
                sw = wv[k]
                for g in range(NG):
                    sl = pl.ds(g * _LANES, _LANES)
                    rowsd_v[r, sl] = rowsd_v[r, sl] * sw
            return carry

        lax.fori_loop(0, DPW // _LANES, dbody, 0)
        pltpu.sync_copy(rowsd_v, bag_hbm.at[pl.ds(w * DPW, DPW)])

        def accumulate(j, buf, accs):
            def rbody(rg, a):
                wv = wt_v[j, pl.ds(rg * _LANES, _LANES)]
                for kk in range(_LANES):
                    r = rg * _LANES + kk
                    sw = wv[kk]
                    a = tuple(
                        a[g] + buf[r, pl.ds(g * _LANES, _LANES)] * sw
                        for g in range(NG)
                    )
                return a

            return lax.fori_loop(0, SUB // _LANES, rbody, accs)

        zero = jnp.zeros((_LANES,), jnp.float32)
        accs = (zero,) * NG
        for j in range(NSUB):
            nxt = None
            if j + 1 < NSUB:
                nxt = pltpu.async_copy(
                    emb_hbm.at[idxt_v.at[j + 1]],
                    bufs[(j + 1) % 2], sems[(j + 1) % 2])
            pending.wait()
            accs = accumulate(j, bufs[j % 2], accs)
            pending = nxt
        for g in range(NG):
            acc_v[0, pl.ds(g * _LANES, _LANES)] = accs[g]
        pltpu.sync_copy(acc_v, part_hbm.at[pl.ds(w, 1)])

    return k(idx_d, w_d, idx_t, w_t, emb)


def _tc_dense(bag, partials, l1_w, l1_b, l2_w, l2_b, l3_w, l3_b):
    B, H = bag.shape
    V = l3_w.shape[0]
    VB = 4096
    nblk = V // VB
    VFULL = nblk * VB

    def body(bag_ref, part_ref, l1w_ref, l1b_ref, l2w_ref, l2b_ref,
             l3w_ref, l3b_ref, out_hbm, z_out, z_ref, buf_ref, sem):
        i = pl.program_id(0)
        p = lax.rem(i, 2)

        @pl.when(i == 0)
        def _():
            rows = lax.broadcasted_iota(jnp.int32, (B, 1), 0)
            tail = jnp.sum(part_ref[...], axis=0, keepdims=True)
            bagf = bag_ref[...] + jnp.where(rows == B - 1, 1.0, 0.0) * tail
            x = jnp.tanh(bagf)
            h = jnp.tanh(
                lax.dot_general(x, l1w_ref[...], (((1,), (1,)), ((), ())),
                                preferred_element_type=jnp.float32)
                + l1b_ref[...])
            z = jnp.tanh(
                lax.dot_general(h, l2w_ref[...], (((1,), (1,)), ((), ())),
                                preferred_element_type=jnp.float32)
                + l2b_ref[...])
            z_ref[...] = z
            z_out[...] = z

        def tile_copy(buf_slot, blk):
            return pltpu.make_async_copy(
                buf_ref.at[buf_slot],
                out_hbm.at[:, pl.ds(blk * VB, VB)],
                sem.at[buf_slot])

        @pl.when(i >= 2)
        def _():
            tile_copy(p, i - 2).wait()

        buf_ref[p] = (
            lax.dot_general(z_ref[...], l3w_ref[...], (((1,), (1,)), ((), ())),
                            preferred_element_type=jnp.float32)
            + l3b_ref[0])
        tile_copy(p, i).start()

        @pl.when(i == nblk - 1)
        def _():
            tile_copy(1 - p, i - 1).wait()
            tile_copy(p, i).wait()

    main, z = pl.pallas_call(
        body,
        grid=(nblk,),
        in_specs=[
            pl.BlockSpec((B, H), lambda i: (0, 0)),
            pl.BlockSpec(partials.shape, lambda i: (0, 0)),
            pl.BlockSpec(l1_w.shape, lambda i: (0, 0)),
            pl.BlockSpec((1, l1_w.shape[0]), lambda i: (0, 0)),
            pl.BlockSpec(l2_w.shape, lambda i: (0, 0)),
            pl.BlockSpec((1, l2_w.shape[0]), lambda i: (0, 0)),
            pl.BlockSpec((VB, H), lambda i: (i, 0)),
            pl.BlockSpec((1, 1, VB), lambda i: (i, 0, 0)),
        ],
        out_specs=[
            pl.BlockSpec(memory_space=pl.ANY),
            pl.BlockSpec((B, H), lambda i: (0, 0)),
        ],
        out_shape=[
            jax.ShapeDtypeStruct((B, V), jnp.float32),
            jax.ShapeDtypeStruct((B, H), jnp.float32),
        ],
        scratch_shapes=[
            pltpu.VMEM((B, H), jnp.float32),
            pltpu.VMEM((2, B, VB), jnp.float32),
            pltpu.SemaphoreType.DMA((2,)),
        ],
        compiler_params=pltpu.CompilerParams(
            dimension_semantics=("arbitrary",)),
    )(bag, partials, l1_w, l1_b.reshape(1, -1), l2_w, l2_b.reshape(1, -1),
      l3_w, l3_b[:VFULL].reshape(nblk, 1, VB))

    PW = 2048
    pblk = VFULL // PW
    l3w_tail = jnp.pad(l3_w[VFULL:], ((0, PW - (V - VFULL)), (0, 0)))
    l3b_tail = jnp.pad(l3_b[VFULL:], (0, PW - (V - VFULL))).reshape(1, PW)

    def patch_body(m_ref, z_ref, w_ref, b_ref, out_ref):
        del m_ref
        out_ref[...] = (
            lax.dot_general(z_ref[...], w_ref[...], (((1,), (1,)), ((), ())),
                            preferred_element_type=jnp.float32)
            + b_ref[...])

    return pl.pallas_call(
        patch_body,
        grid=(1,),
        in_specs=[
            pl.BlockSpec(memory_space=pl.ANY),
            pl.BlockSpec((B, H), lambda i: (0, 0)),
            pl.BlockSpec((PW, H), lambda i: (0, 0)),
            pl.BlockSpec((1, PW), lambda i: (0, 0)),
        ],
        out_specs=pl.BlockSpec((B, PW), lambda i: (0, pblk)),
        out_shape=jax.ShapeDtypeStruct((B, V), jnp.float32),
        input_output_aliases={0: 0},
    )(main, z, l3w_tail, l3b_tail)


def kernel(array, offsets, weights, emb_w, l1_w, l1_b, l2_w, l2_b, l3_w, l3_b):
    B = offsets.shape[0]
    N = array.shape[0]
    NW = 32
    DPW = B // NW
    tail = N - B
    per_w = tail // NW
    SUB = 112
    NSUB = per_w // SUB

    arr = array.astype(jnp.int32)
    idx_d = arr[:B].reshape(NW, DPW)
    w_d = weights[:B].reshape(NW, DPW)
    idx_t = arr[B:].reshape(NW, NSUB, SUB)
    w_t = weights[B:].reshape(NW, NSUB, SUB)

    bag, partials = _sc_embedding_bag(idx_d, w_d, idx_t, w_t, emb_w)
    return _tc_dense(bag, partials, l1_w, l1_b, l2_w, l2_b, l3_w, l3_b)


# speedup vs baseline: 1.1018x; 1.0011x over previous
import functools

import jax
import jax.numpy as jnp
from jax import lax
from jax.experimental import pallas as pl
from jax.experimental.pallas import tpu as pltpu
from jax.experimental.pallas import tpu_sc as plsc

_LANES = 16


def _sc_embedding_bag(idx_d, w_d, idx_t, w_t, emb):
    NW, DPW = idx_d.shape
    _, NSUB, SUB = idx_t.shape
    V, H = emb.shape
    NG = H // _LANES
    mesh = plsc.VectorSubcoreMesh(core_axis_name="c", subcore_axis_name="s")
    NC = 2

    @functools.partial(
        pl.kernel,
        mesh=mesh,
        out_type=[
            jax.ShapeDtypeStruct((NW * DPW, H), jnp.float32),
            jax.ShapeDtypeStruct((NW, H), jnp.float32),
        ],
        scratch_types=[
            pltpu.VMEM((DPW,), jnp.int32),
            pltpu.VMEM((DPW,), jnp.float32),
            pltpu.VMEM((NSUB, SUB), jnp.int32),
            pltpu.VMEM((NSUB, SUB), jnp.float32),
            pltpu.VMEM((DPW, H), jnp.float32),
            pltpu.VMEM((SUB, H), jnp.float32),
            pltpu.VMEM((SUB, H), jnp.float32),
            pltpu.VMEM((1, H), jnp.float32),
            pltpu.SemaphoreType.DMA,
            pltpu.SemaphoreType.DMA,
        ],
    )
    def k(idx_d_hbm, w_d_hbm, idx_t_hbm, w_t_hbm, emb_hbm, bag_hbm, part_hbm,
          idxd_v, wd_v, idxt_v, wt_v, rowsd_v, rows0_v, rows1_v, acc_v,
          sem0, sem1):
        c = lax.axis_index("c")
        s = lax.axis_index("s")
        w = s * NC + c

        c0 = pltpu.async_copy(idx_t_hbm.at[w], idxt_v, sem0)
        c1 = pltpu.async_copy(w_t_hbm.at[w], wt_v, sem0)
        c2 = pltpu.async_copy(idx_d_hbm.at[w], idxd_v, sem1)
        c3 = pltpu.async_copy(w_d_hbm.at[w], wd_v, sem1)
        c2.wait()
        c3.wait()
        c0.wait()
        c1.wait()

        bufs = (rows0_v, rows1_v)
        sems = (sem0, sem1)
        pending = pltpu.async_copy(emb_hbm.at[idxt_v.at[0]], bufs[0], sems[0])

        pltpu.async_copy(emb_hbm.at[idxd_v], rowsd_v, sem1).wait()

        def dbody(rg, carry):
            wv = wd_v[pl.ds(rg * _LANES, _LANES)]
            for k in range(_LANES):
                r = rg * _LANES + k
                sw = wv[k]
                for g in range(NG):
                    sl = pl.ds(g * _LANES, _LANES)
                    rowsd_v[r, sl] = rowsd_v[r, sl] * sw
            return carry

        lax.fori_loop(0, DPW // _LANES, dbody, 0)
        pltpu.sync_copy(rowsd_v, bag_hbm.at[pl.ds(w * DPW, DPW)])

        def accumulate(j, buf, accs):
            def rbody(rg, a):
                wv = wt_v[j, pl.ds(rg * _LANES, _LANES)]
                for kk in range(_LANES):
                    r = rg * _LANES + kk
                    sw = wv[kk]
                    a = tuple(
                        a[g] + buf[r, pl.ds(g * _LANES, _LANES)] * sw
                        for g in range(NG)
                    )
                return a

            return lax.fori_loop(0, SUB // _LANES, rbody, accs)

        zero = jnp.zeros((_LANES,), jnp.float32)
        accs = (zero,) * NG
        for j in range(NSUB):
            nxt = None
            if j + 1 < NSUB:
                nxt = pltpu.async_copy(
                    emb_hbm.at[idxt_v.at[j + 1]],
                    bufs[(j + 1) % 2], sems[(j + 1) % 2])
            pending.wait()
            accs = accumulate(j, bufs[j % 2], accs)
            pending = nxt
        for g in range(NG):
            acc_v[0, pl.ds(g * _LANES, _LANES)] = accs[g]
        pltpu.sync_copy(acc_v, part_hbm.at[pl.ds(w, 1)])

    return k(idx_d, w_d, idx_t, w_t, emb)


def _tc_dense(bag, partials, l1_w, l1_b, l2_w, l2_b, l3_w, l3_b):
    B, H = bag.shape
    V = l3_w.shape[0]
    VB = 6144
    nblk = V // VB
    VFULL = nblk * VB

    def body(bag_ref, part_ref, l1w_ref, l1b_ref, l2w_ref, l2b_ref,
             l3w_ref, l3b_ref, out_hbm, z_out, z_ref, buf_ref, sem):
        i = pl.program_id(0)
        p = lax.rem(i, 2)

        @pl.when(i == 0)
        def _():
            rows = lax.broadcasted_iota(jnp.int32, (B, 1), 0)
            tail = jnp.sum(part_ref[...], axis=0, keepdims=True)
            bagf = bag_ref[...] + jnp.where(rows == B - 1, 1.0, 0.0) * tail
            x = jnp.tanh(bagf)
            h = jnp.tanh(
                lax.dot_general(x, l1w_ref[...], (((1,), (1,)), ((), ())),
                                preferred_element_type=jnp.float32)
                + l1b_ref[...])
            z = jnp.tanh(
                lax.dot_general(h, l2w_ref[...], (((1,), (1,)), ((), ())),
                                preferred_element_type=jnp.float32)
                + l2b_ref[...])
            z_ref[...] = z
            z_out[...] = z

        def tile_copy(buf_slot, blk):
            return pltpu.make_async_copy(
                buf_ref.at[buf_slot],
                out_hbm.at[:, pl.ds(blk * VB, VB)],
                sem.at[buf_slot])

        @pl.when(i >= 2)
        def _():
            tile_copy(p, i - 2).wait()

        buf_ref[p] = (
            lax.dot_general(z_ref[...], l3w_ref[...], (((1,), (1,)), ((), ())),
                            preferred_element_type=jnp.float32)
            + l3b_ref[0])
        tile_copy(p, i).start()

        @pl.when(i == nblk - 1)
        def _():
            tile_copy(1 - p, i - 1).wait()
            tile_copy(p, i).wait()

    main, z = pl.pallas_call(
        body,
        grid=(nblk,),
        in_specs=[
            pl.BlockSpec((B, H), lambda i: (0, 0)),
            pl.BlockSpec(partials.shape, lambda i: (0, 0)),
            pl.BlockSpec(l1_w.shape, lambda i: (0, 0)),
            pl.BlockSpec((1, l1_w.shape[0]), lambda i: (0, 0)),
            pl.BlockSpec(l2_w.shape, lambda i: (0, 0)),
            pl.BlockSpec((1, l2_w.shape[0]), lambda i: (0, 0)),
            pl.BlockSpec((VB, H), lambda i: (i, 0)),
            pl.BlockSpec((1, 1, VB), lambda i: (i, 0, 0)),
        ],
        out_specs=[
            pl.BlockSpec(memory_space=pl.ANY),
            pl.BlockSpec((B, H), lambda i: (0, 0)),
        ],
        out_shape=[
            jax.ShapeDtypeStruct((B, V), jnp.float32),
            jax.ShapeDtypeStruct((B, H), jnp.float32),
        ],
        scratch_shapes=[
            pltpu.VMEM((B, H), jnp.float32),
            pltpu.VMEM((2, B, VB), jnp.float32),
            pltpu.SemaphoreType.DMA((2,)),
        ],
        compiler_params=pltpu.CompilerParams(
            dimension_semantics=("arbitrary",)),
    )(bag, partials, l1_w, l1_b.reshape(1, -1), l2_w, l2_b.reshape(1, -1),
      l3_w, l3_b[:VFULL].reshape(nblk, 1, VB))

    PW = 2048
    pblk = VFULL // PW
    l3w_tail = jnp.pad(l3_w[VFULL:], ((0, PW - (V - VFULL)), (0, 0)))
    l3b_tail = jnp.pad(l3_b[VFULL:], (0, PW - (V - VFULL))).reshape(1, PW)

    def patch_body(m_ref, z_ref, w_ref, b_ref, out_ref):
        del m_ref
        out_ref[...] = (
            lax.dot_general(z_ref[...], w_ref[...], (((1,), (1,)), ((), ())),
                            preferred_element_type=jnp.float32)
            + b_ref[...])

    return pl.pallas_call(
        patch_body,
        grid=(1,),
        in_specs=[
            pl.BlockSpec(memory_space=pl.ANY),
            pl.BlockSpec((B, H), lambda i: (0, 0)),
            pl.BlockSpec((PW, H), lambda i: (0, 0)),
            pl.BlockSpec((1, PW), lambda i: (0, 0)),
        ],
        out_specs=pl.BlockSpec((B, PW), lambda i: (0, pblk)),
        out_shape=jax.ShapeDtypeStruct((B, V), jnp.float32),
        input_output_aliases={0: 0},
    )(main, z, l3w_tail, l3b_tail)


def kernel(array, offsets, weights, emb_w, l1_w, l1_b, l2_w, l2_b, l3_w, l3_b):
    B = offsets.shape[0]
    N = array.shape[0]
    NW = 32
    DPW = B // NW
    tail = N - B
    per_w = tail // NW
    SUB = 112
    NSUB = per_w // SUB

    arr = array.astype(jnp.int32)
    idx_d = arr[:B].reshape(NW, DPW)
    w_d = weights[:B].reshape(NW, DPW)
    idx_t = arr[B:].reshape(NW, NSUB, SUB)
    w_t = weights[B:].reshape(NW, NSUB, SUB)

    bag, partials = _sc_embedding_bag(idx_d, w_d, idx_t, w_t, emb_w)
    return _tc_dense(bag, partials, l1_w, l1_b, l2_w, l2_b, l3_w, l3_b)
